# baseline (speedup 1.0000x reference)
import functools

import jax
import jax.numpy as jnp
from jax import lax
from jax.experimental import pallas as pl
from jax.experimental.pallas import tpu as pltpu
from jax.experimental.pallas import tpu_sc as plsc

_B = 4096
_DU = 128
_DA = 32
_KA = 1000
_TB = 512


@functools.lru_cache(maxsize=None)
def _build_sc():
    info = plsc.get_sparse_core_info()
    nc, ns = info.num_cores, info.num_subcores
    nw = nc * ns
    bpw = _B // nw

    mesh = plsc.VectorSubcoreMesh(core_axis_name="c", subcore_axis_name="s")

    @functools.partial(
        pl.kernel,
        mesh=mesh,
        compiler_params=pltpu.CompilerParams(needs_layout_passes=False),
        out_type=(
            jax.ShapeDtypeStruct((_B, _DU), jnp.float32),
            jax.ShapeDtypeStruct((_B, _DU), jnp.float32),
        ),
        scratch_types=[
            pltpu.VMEM((bpw,), jnp.int32),
            pltpu.VMEM((bpw,), jnp.int32),
            pltpu.VMEM((bpw, _DU), jnp.float32),
            pltpu.VMEM((bpw, _DU), jnp.float32),
            pltpu.SemaphoreType.DMA,
            pltpu.SemaphoreType.DMA,
            pltpu.SemaphoreType.DMA,
            pltpu.SemaphoreType.DMA,
        ],
    )
    def emb(user_hbm, item_hbm, un_hbm, in_hbm,
            u_out, i_out,
            uidx, iidx, urows, irows,
            su, si, syu, syi):
        wid = lax.axis_index("s") * nc + lax.axis_index("c")
        base = wid * bpw
        gu = pltpu.async_copy(un_hbm.at[pl.ds(base, bpw)], uidx, syu)
        gi = pltpu.async_copy(in_hbm.at[pl.ds(base, bpw)], iidx, syi)
        gu.wait()
        cu = pltpu.async_copy(user_hbm.at[uidx], urows, su)
        gi.wait()
        ci = pltpu.async_copy(item_hbm.at[iidx], irows, si)
        cu.wait()
        pltpu.sync_copy(urows, u_out.at[pl.ds(base, bpw)])
        ci.wait()
        pltpu.sync_copy(irows, i_out.at[pl.ds(base, bpw)])

    return emb


def _attr_body(idx_ref, tab_ref, o_ref):
    idx = idx_ref[:]
    k_iota = lax.broadcasted_iota(jnp.int32, (_TB, _KA), 1)
    one_hot = (k_iota == idx[:, None]).astype(jnp.float32)
    o_ref[:] = jnp.dot(one_hot, tab_ref[:],
                       preferred_element_type=jnp.float32,
                       precision=lax.Precision.HIGHEST)


@functools.lru_cache(maxsize=None)
def _build_attr_tc():
    return pl.pallas_call(
        _attr_body,
        grid=(_B // _TB,),
        in_specs=[
            pl.BlockSpec((_TB,), lambda i: (i,)),
            pl.BlockSpec((_KA, _DA), lambda i: (0, 0)),
        ],
        out_specs=pl.BlockSpec((_TB, _DA), lambda i: (i, 0)),
        out_shape=jax.ShapeDtypeStruct((_B, _DA), jnp.float32),
    )


def kernel(user_table, item_table, attr_table, user_nodes, item_nodes,
           attribute_nodes):
    emb = _build_sc()
    attr_tc = _build_attr_tc()
    u, i = emb(
        user_table, item_table,
        user_nodes.astype(jnp.int32),
        item_nodes.astype(jnp.int32),
    )
    at = attr_tc(attribute_nodes.astype(jnp.int32), attr_table)
    return (u, i, at)


# speedup vs baseline: 1.3019x; 1.3019x over previous
import functools

import jax
import jax.numpy as jnp
from jax import lax
from jax.experimental import pallas as pl
from jax.experimental.pallas import tpu as pltpu
from jax.experimental.pallas import tpu_sc as plsc

_B = 4096
_DU = 128
_DA = 32
_KA = 1000
_KW = 250
_TB = 512


@functools.lru_cache(maxsize=None)
def _build_sc():
    info = plsc.get_sparse_core_info()
    nc, ns = info.num_cores, info.num_subcores
    nw = nc * ns
    bpw = _B // nw

    mesh = plsc.VectorSubcoreMesh(core_axis_name="c", subcore_axis_name="s")

    @functools.partial(
        pl.kernel,
        mesh=mesh,
        compiler_params=pltpu.CompilerParams(needs_layout_passes=False),
        out_type=(
            jax.ShapeDtypeStruct((_B, _DU), jnp.float32),
            jax.ShapeDtypeStruct((_B, _DU), jnp.float32),
        ),
        scratch_types=[
            pltpu.VMEM((bpw,), jnp.int32),
            pltpu.VMEM((bpw,), jnp.int32),
            pltpu.VMEM((bpw, _DU), jnp.float32),
            pltpu.VMEM((bpw, _DU), jnp.float32),
            pltpu.SemaphoreType.DMA,
            pltpu.SemaphoreType.DMA,
            pltpu.SemaphoreType.DMA,
            pltpu.SemaphoreType.DMA,
        ],
    )
    def emb(user_hbm, item_hbm, un_hbm, in_hbm,
            u_out, i_out,
            uidx, iidx, urows, irows,
            su, si, syu, syi):
        wid = lax.axis_index("s") * nc + lax.axis_index("c")
        base = wid * bpw
        gu = pltpu.async_copy(un_hbm.at[pl.ds(base, bpw)], uidx, syu)
        gi = pltpu.async_copy(in_hbm.at[pl.ds(base, bpw)], iidx, syi)
        gu.wait()
        cu = pltpu.async_copy(user_hbm.at[uidx], urows, su)
        gi.wait()
        ci = pltpu.async_copy(item_hbm.at[iidx], irows, si)
        cu.wait()
        pltpu.sync_copy(urows, u_out.at[pl.ds(base, bpw)])
        ci.wait()
        pltpu.sync_copy(irows, i_out.at[pl.ds(base, bpw)])

    return emb


def _attr_body(idx_ref, tabw_ref, o_ref):
    idx = idx_ref[:]
    wrow = lax.shift_right_logical(idx, 2)
    sub = idx & 3
    k_iota = lax.broadcasted_iota(jnp.int32, (_TB, _KW), 1)
    one_hot = (k_iota == wrow[:, None]).astype(jnp.float32)
    wide = jnp.dot(one_hot, tabw_ref[:],
                   preferred_element_type=jnp.float32,
                   precision=lax.Precision.HIGHEST)
    acc = jnp.zeros((_TB, _DA), jnp.float32)
    for q in range(4):
        m = (sub == q).astype(jnp.float32)[:, None]
        acc = acc + m * wide[:, q * _DA:(q + 1) * _DA]
    o_ref[:] = acc


@functools.lru_cache(maxsize=None)
def _build_attr_tc():
    return pl.pallas_call(
        _attr_body,
        grid=(_B // _TB,),
        in_specs=[
            pl.BlockSpec((_TB,), lambda i: (i,)),
            pl.BlockSpec((_KW, _DU), lambda i: (0, 0)),
        ],
        out_specs=pl.BlockSpec((_TB, _DA), lambda i: (i, 0)),
        out_shape=jax.ShapeDtypeStruct((_B, _DA), jnp.float32),
    )


def kernel(user_table, item_table, attr_table, user_nodes, item_nodes,
           attribute_nodes):
    emb = _build_sc()
    attr_tc = _build_attr_tc()
    u, i = emb(
        user_table, item_table,
        user_nodes.astype(jnp.int32),
        item_nodes.astype(jnp.int32),
    )
    at = attr_tc(attribute_nodes.astype(jnp.int32),
                 attr_table.reshape(_KW, _DU))
    return (u, i, at)


# speedup vs baseline: 1.5691x; 1.2053x over previous
import functools

import jax
import jax.numpy as jnp
from jax import lax
from jax.experimental import pallas as pl
from jax.experimental.pallas import tpu as pltpu
from jax.experimental.pallas import tpu_sc as plsc

_B = 4096
_DU = 128
_DA = 32
_KA = 1000
_KW = 250
_TB = 512


@functools.lru_cache(maxsize=None)
def _build_sc():
    info = plsc.get_sparse_core_info()
    nc, ns = info.num_cores, info.num_subcores
    nw = nc * ns
    bpw = _B // nw

    mesh = plsc.VectorSubcoreMesh(core_axis_name="c", subcore_axis_name="s")

    @functools.partial(
        pl.kernel,
        mesh=mesh,
        compiler_params=pltpu.CompilerParams(needs_layout_passes=False),
        out_type=(
            jax.ShapeDtypeStruct((_B, _DU), jnp.float32),
            jax.ShapeDtypeStruct((_B, _DU), jnp.float32),
        ),
        scratch_types=[
            pltpu.VMEM((bpw,), jnp.int32),
            pltpu.VMEM((bpw,), jnp.int32),
            pltpu.VMEM((bpw, _DU), jnp.float32),
            pltpu.VMEM((bpw, _DU), jnp.float32),
            pltpu.SemaphoreType.DMA,
            pltpu.SemaphoreType.DMA,
            pltpu.SemaphoreType.DMA,
            pltpu.SemaphoreType.DMA,
        ],
    )
    def emb(user_hbm, item_hbm, un_hbm, in_hbm,
            u_out, i_out,
            uidx, iidx, urows, irows,
            su, si, syu, syi):
        wid = lax.axis_index("s") * nc + lax.axis_index("c")
        base = wid * bpw
        gu = pltpu.async_copy(un_hbm.at[pl.ds(base, bpw)], uidx, syu)
        gi = pltpu.async_copy(in_hbm.at[pl.ds(base, bpw)], iidx, syi)
        gu.wait()
        cu = pltpu.async_copy(user_hbm.at[uidx], urows, su)
        gi.wait()
        ci = pltpu.async_copy(item_hbm.at[iidx], irows, si)
        cu.wait()
        pltpu.sync_copy(urows, u_out.at[pl.ds(base, bpw)])
        ci.wait()
        pltpu.sync_copy(irows, i_out.at[pl.ds(base, bpw)])

    return emb


def _attr_body(idx_ref, tabw_ref, o_ref):
    idx = idx_ref[:]
    wrow = lax.shift_right_logical(idx, 2)
    sub = idx & 3
    k_iota = lax.broadcasted_iota(jnp.int32, (_KW, _B), 0)
    one_hot_t = (k_iota == wrow[None, :]).astype(jnp.float32)
    wide_t = jnp.dot(tabw_ref[:].T, one_hot_t,
                     preferred_element_type=jnp.float32,
                     precision=lax.Precision.HIGHEST)
    acc = jnp.zeros((_DA, _B), jnp.float32)
    for q in range(4):
        m = (sub == q).astype(jnp.float32)[None, :]
        acc = acc + m * wide_t[q * _DA:(q + 1) * _DA, :]
    o_ref[:] = acc


@functools.lru_cache(maxsize=None)
def _build_attr_tc():
    return pl.pallas_call(
        _attr_body,
        out_shape=jax.ShapeDtypeStruct((_DA, _B), jnp.float32),
    )


def kernel(user_table, item_table, attr_table, user_nodes, item_nodes,
           attribute_nodes):
    emb = _build_sc()
    attr_tc = _build_attr_tc()
    u, i = emb(
        user_table, item_table,
        user_nodes.astype(jnp.int32),
        item_nodes.astype(jnp.int32),
    )
    at = attr_tc(attribute_nodes.astype(jnp.int32),
                 attr_table.reshape(_KW, _DU))
    return (u, i, at.T)
